# Initial kernel scaffold; baseline (speedup 1.0000x reference)
import jax
import jax.numpy as jnp
from jax.experimental import pallas as pl


def kernel(x, edge_index, W1l, b1l, W1r, W2l, b2l, W2r, W3l, b3l, W3r, Wlin, blin):
    raise NotImplementedError("write your pallas kernel here")


# speedup vs baseline: 1.7075x; 1.7075x over previous
import functools

import jax
import jax.numpy as jnp
from jax import lax
from jax.experimental import pallas as pl
from jax.experimental.pallas import tpu as pltpu
from jax.experimental.pallas import tpu_sc as plsc

N = 10000
F_IN = 256
H = 512
C = 16
E = 160000

FC = 128
NROWS = 10112
ROWS_PT = NROWS // 16
DUMP = N
CHUNK = 128
CHUNKS_PT = 40
EP = 32 * CHUNKS_PT * CHUNK
NCHUNKS = EP // CHUNK
CW = 16
R = 400


def _sc_agg(npass, with_counts):
    mesh = plsc.VectorSubcoreMesh(core_axis_name="c", subcore_axis_name="s",
                                  num_cores=2, num_subcores=16)
    outs = [jax.ShapeDtypeStruct((2, NROWS, FC), jnp.float32) for _ in range(npass)]
    if with_counts:
        outs.append(jax.ShapeDtypeStruct((2, NROWS, CW), jnp.float32))
    scratch = [
        pltpu.VMEM((CHUNK,), jnp.int32),
        pltpu.VMEM((CHUNK,), jnp.int32),
        pltpu.VMEM((CHUNK, FC), jnp.float32),
        pltpu.VMEM((CHUNK, FC), jnp.float32),
        pltpu.VMEM_SHARED((NROWS, FC), jnp.float32),
        pltpu.SemaphoreType.DMA,
    ]
    if with_counts:
        scratch += [
            pltpu.VMEM((CHUNK, CW), jnp.float32),
            pltpu.VMEM((CHUNK, CW), jnp.float32),
            pltpu.VMEM_SHARED((NROWS, CW), jnp.float32),
        ]

    def body(*refs):
        xs = refs[:npass]
        src2d, dst2d = refs[npass], refs[npass + 1]
        o = npass + 2
        outs_r = refs[o:o + npass]
        o += npass
        if with_counts:
            cnt_out = refs[o]
            o += 1
        src_idx, dst_idx, rows, zbuf, acc, sem = refs[o:o + 6]
        o += 6
        if with_counts:
            ones, zbuf16, cnt_acc = refs[o:o + 3]

        ci = lax.axis_index("c")
        si = lax.axis_index("s")
        tile0 = (ci * 16 + si) * CHUNKS_PT
        base = si * ROWS_PT

        zero16 = jnp.zeros((16,), jnp.float32)
        one16 = jnp.ones((16,), jnp.float32)

        @pl.loop(0, CHUNK)
        def _fill(i):
            @pl.loop(0, FC // 16)
            def _fz(j):
                zbuf[i, pl.ds(j * 16, 16)] = zero16
            if with_counts:
                ones[i, pl.ds(0, 16)] = one16
                zbuf16[i, pl.ds(0, 16)] = zero16

        if with_counts:
            for k, sz in ((0, 128), (128, 128), (256, 128), (384, 128), (512, 120)):
                pltpu.sync_copy(zbuf16.at[pl.ds(0, sz)],
                                cnt_acc.at[pl.ds(base + k, sz)])

        for p in range(npass):
            for k, sz in ((0, 128), (128, 128), (256, 128), (384, 128), (512, 120)):
                pltpu.sync_copy(zbuf.at[pl.ds(0, sz)],
                                acc.at[pl.ds(base + k, sz)])
            plsc.subcore_barrier()

            @pl.loop(0, CHUNKS_PT)
            def _chunk(j):
                g = tile0 + j
                pltpu.sync_copy(src2d.at[g], src_idx)
                pltpu.sync_copy(dst2d.at[g], dst_idx)
                pltpu.async_copy(xs[p].at[src_idx], rows, sem).wait()
                pltpu.sync_copy(rows, acc.at[dst_idx], add=True)
                if with_counts and p == 0:
                    pltpu.sync_copy(ones, cnt_acc.at[dst_idx], add=True)

            plsc.subcore_barrier()
            pltpu.sync_copy(acc.at[pl.ds(base, ROWS_PT)],
                            outs_r[p].at[ci, pl.ds(base, ROWS_PT)])
            if with_counts and p == 0:
                pltpu.sync_copy(cnt_acc.at[pl.ds(base, ROWS_PT)],
                                cnt_out.at[ci, pl.ds(base, ROWS_PT)])

    return pl.kernel(body, out_type=outs, mesh=mesh, scratch_types=scratch,
                     compiler_params=pltpu.CompilerParams(use_tc_tiling_on_sc=False))


def _dense_layer(npass_in, f_in, final):
    grid = (N // R,)

    in_specs = (
        [pl.BlockSpec((2, R, FC), lambda i: (0, i, 0)) for _ in range(npass_in)]
        + [pl.BlockSpec((2, R, CW), lambda i: (0, i, 0))]
        + [pl.BlockSpec((R, FC), lambda i: (i, 0)) for _ in range(npass_in)]
        + [pl.BlockSpec((f_in, H), lambda i: (0, 0)),
           pl.BlockSpec((f_in, H), lambda i: (0, 0)),
           pl.BlockSpec((1, H), lambda i: (0, 0))]
    )
    if final:
        in_specs += (
            [pl.BlockSpec((R, FC), lambda i: (i, 0)) for _ in range(8)]
            + [pl.BlockSpec((3 * H, C), lambda i: (0, 0)),
               pl.BlockSpec((1, C), lambda i: (0, 0))]
        )
        out_specs = pl.BlockSpec((R, C), lambda i: (i, 0))
        out_shape = jax.ShapeDtypeStruct((N, C), jnp.float32)
    else:
        out_specs = [pl.BlockSpec((R, FC), lambda i: (i, 0)) for _ in range(H // FC)]
        out_shape = [jax.ShapeDtypeStruct((N, FC), jnp.float32) for _ in range(H // FC)]

    def body(*refs):
        parts = refs[:npass_in]
        cnt = refs[npass_in]
        o = npass_in + 1
        xins = refs[o:o + npass_in]
        o += npass_in
        Wl, Wr, b = refs[o:o + 3]
        o += 3
        if final:
            hprev = refs[o:o + 8]
            Wlin, blin = refs[o + 8:o + 10]
            o += 10
            out = refs[o]
        else:
            outs = refs[o:]

        s = [pr[0] + pr[1] for pr in parts]
        mean = jnp.concatenate(s, axis=1)
        cb = cnt[...]
        deg = cb[0, :, 0:1] + cb[1, :, 0:1]
        mean = mean / jnp.maximum(deg, 1.0)
        x = jnp.concatenate([xr[...] for xr in xins], axis=1)
        h = (jnp.dot(mean, Wl[...], preferred_element_type=jnp.float32)
             + jnp.dot(x, Wr[...], preferred_element_type=jnp.float32)
             + b[...])
        nrm = jnp.sqrt(jnp.sum(h * h, axis=1, keepdims=True))
        h = jnp.maximum(h / jnp.maximum(nrm, 1e-12), 0.0)
        if final:
            cat = jnp.concatenate([hr[...] for hr in hprev] + [h], axis=1)
            out[...] = (jnp.dot(cat, Wlin[...], preferred_element_type=jnp.float32)
                        + blin[...])
        else:
            for q in range(H // FC):
                outs[q][...] = h[:, q * FC:(q + 1) * FC]

    return pl.pallas_call(body, grid=grid, in_specs=in_specs,
                          out_specs=out_specs, out_shape=out_shape)


_AGG1 = _sc_agg(F_IN // FC, with_counts=True)
_AGG2 = _sc_agg(H // FC, with_counts=False)
_DENSE1 = _dense_layer(F_IN // FC, F_IN, final=False)
_DENSE2 = _dense_layer(H // FC, H, final=False)
_DENSE3 = _dense_layer(H // FC, H, final=True)


def kernel(x, edge_index, W1l, b1l, W1r, W2l, b2l, W2r, W3l, b3l, W3r, Wlin, blin):
    src = edge_index[0].astype(jnp.int32)
    dst = edge_index[1].astype(jnp.int32)
    pad = EP - E
    srcp = jnp.concatenate([src, jnp.zeros((pad,), jnp.int32)]).reshape(NCHUNKS, CHUNK)
    dstp = jnp.concatenate([dst, jnp.full((pad,), DUMP, jnp.int32)]).reshape(NCHUNKS, CHUNK)
    x_c = [x[:, i * FC:(i + 1) * FC] for i in range(F_IN // FC)]
    b1 = b1l.reshape(1, H)
    b2 = b2l.reshape(1, H)
    b3 = b3l.reshape(1, H)
    bl = blin.reshape(1, C)

    p10, p11, cnt = _AGG1(x_c[0], x_c[1], srcp, dstp)
    h1c = _DENSE1(p10, p11, cnt, x_c[0], x_c[1], W1l, W1r, b1)
    p2 = _AGG2(h1c[0], h1c[1], h1c[2], h1c[3], srcp, dstp)
    h2c = _DENSE2(p2[0], p2[1], p2[2], p2[3], cnt,
                  h1c[0], h1c[1], h1c[2], h1c[3], W2l, W2r, b2)
    p3 = _AGG2(h2c[0], h2c[1], h2c[2], h2c[3], srcp, dstp)
    final = _DENSE3(p3[0], p3[1], p3[2], p3[3], cnt,
                    h2c[0], h2c[1], h2c[2], h2c[3], W3l, W3r, b3,
                    h1c[0], h1c[1], h1c[2], h1c[3],
                    h2c[0], h2c[1], h2c[2], h2c[3], Wlin, bl)
    return final


# speedup vs baseline: 2.0449x; 1.1975x over previous
import functools

import jax
import jax.numpy as jnp
from jax import lax
from jax.experimental import pallas as pl
from jax.experimental.pallas import tpu as pltpu
from jax.experimental.pallas import tpu_sc as plsc

N = 10000
F_IN = 256
H = 512
C = 16
E = 160000

FC = 128
NROWS = 10112
ROWS_PT = NROWS // 16
DUMP = N
CH = 64
NCH = 80
EP = 32 * NCH * CH
NCHUNKS = EP // CH
CW = 16
R = 400
_ZSTEPS = tuple((k, 64) for k in range(0, 576, 64)) + ((576, 56),)


def _sc_agg(npass, with_counts):
    mesh = plsc.VectorSubcoreMesh(core_axis_name="c", subcore_axis_name="s",
                                  num_cores=2, num_subcores=16)
    outs = [jax.ShapeDtypeStruct((2, NROWS, FC), jnp.float32) for _ in range(npass)]
    if with_counts:
        outs.append(jax.ShapeDtypeStruct((2, NROWS, CW), jnp.float32))
    scratch = [
        pltpu.VMEM((NCH, CH), jnp.int32),
        pltpu.VMEM((NCH, CH), jnp.int32),
        pltpu.VMEM((CH, FC), jnp.float32),
        pltpu.VMEM((CH, FC), jnp.float32),
        pltpu.VMEM((64, FC), jnp.float32),
        pltpu.VMEM_SHARED((NROWS, FC), jnp.float32),
        pltpu.SemaphoreType.DMA,
        pltpu.SemaphoreType.DMA,
        pltpu.SemaphoreType.DMA,
        pltpu.SemaphoreType.DMA,
    ]
    if with_counts:
        scratch += [
            pltpu.VMEM((CH, CW), jnp.float32),
            pltpu.VMEM((64, CW), jnp.float32),
            pltpu.VMEM_SHARED((NROWS, CW), jnp.float32),
            pltpu.SemaphoreType.DMA,
        ]

    def body(*refs):
        xs = refs[:npass]
        src2d, dst2d = refs[npass], refs[npass + 1]
        o = npass + 2
        outs_r = refs[o:o + npass]
        o += npass
        if with_counts:
            cnt_out = refs[o]
            o += 1
        src_v, dst_v, rows0, rows1, zbuf, acc, g0, g1, s0, s1 = refs[o:o + 10]
        o += 10
        if with_counts:
            ones, zbuf16, cnt_acc, semc = refs[o:o + 4]

        ci = lax.axis_index("c")
        si = lax.axis_index("s")
        tid = ci * 16 + si
        base = si * ROWS_PT

        zero16 = jnp.zeros((16,), jnp.float32)
        one16 = jnp.ones((16,), jnp.float32)

        @pl.loop(0, 64)
        def _fill(i):
            @pl.loop(0, FC // 16)
            def _fz(j):
                zbuf[i, pl.ds(j * 16, 16)] = zero16
            if with_counts:
                zbuf16[i, pl.ds(0, 16)] = zero16
        if with_counts:
            @pl.loop(0, CH)
            def _fo(i):
                ones[i, pl.ds(0, 16)] = one16

        pltpu.sync_copy(src2d.at[pl.ds(tid * NCH, NCH)], src_v)
        pltpu.sync_copy(dst2d.at[pl.ds(tid * NCH, NCH)], dst_v)

        if with_counts:
            for k, sz in _ZSTEPS:
                pltpu.sync_copy(zbuf16.at[pl.ds(0, sz)],
                                cnt_acc.at[pl.ds(base + k, sz)])

        for p in range(npass):
            pltpu.async_copy(xs[p].at[src_v.at[0]], rows0, g0)
            for k, sz in _ZSTEPS:
                pltpu.sync_copy(zbuf.at[pl.ds(0, sz)],
                                acc.at[pl.ds(base + k, sz)])
            plsc.subcore_barrier()

            @pl.loop(0, NCH // 2)
            def _chunks(i):
                j0 = 2 * i
                j1 = j0 + 1
                pltpu.async_copy(xs[p].at[src_v.at[j1]], rows1, g1)
                pltpu.make_async_copy(xs[p].at[src_v.at[j0]], rows0, g0).wait()
                pltpu.async_copy(rows0, acc.at[dst_v.at[j0]], s0, add=True)
                if with_counts and p == 0:
                    pltpu.async_copy(ones, cnt_acc.at[dst_v.at[j0]], semc, add=True)
                pltpu.make_async_copy(xs[p].at[src_v.at[j1]], rows1, g1).wait()
                pltpu.async_copy(rows1, acc.at[dst_v.at[j1]], s1, add=True)
                if with_counts and p == 0:
                    pltpu.async_copy(ones, cnt_acc.at[dst_v.at[j1]], semc, add=True)
                pltpu.make_async_copy(rows0, acc.at[dst_v.at[j0]], s0).wait()

                @pl.when(i + 1 < NCH // 2)
                def _pref():
                    pltpu.async_copy(xs[p].at[src_v.at[j0 + 2]], rows0, g0)

                pltpu.make_async_copy(rows1, acc.at[dst_v.at[j1]], s1).wait()

            if with_counts and p == 0:
                @pl.loop(0, NCH)
                def _drain(i):
                    pltpu.make_async_copy(ones, cnt_acc.at[dst_v.at[i]], semc).wait()

            plsc.subcore_barrier()
            pltpu.sync_copy(acc.at[pl.ds(base, ROWS_PT)],
                            outs_r[p].at[ci, pl.ds(base, ROWS_PT)])
            if with_counts and p == 0:
                pltpu.sync_copy(cnt_acc.at[pl.ds(base, ROWS_PT)],
                                cnt_out.at[ci, pl.ds(base, ROWS_PT)])

    return pl.kernel(body, out_type=outs, mesh=mesh, scratch_types=scratch,
                     compiler_params=pltpu.CompilerParams(use_tc_tiling_on_sc=False))


def _dense_layer(npass_in, f_in, final):
    grid = (N // R,)

    in_specs = (
        [pl.BlockSpec((2, R, FC), lambda i: (0, i, 0)) for _ in range(npass_in)]
        + [pl.BlockSpec((2, R, CW), lambda i: (0, i, 0))]
        + [pl.BlockSpec((R, FC), lambda i: (i, 0)) for _ in range(npass_in)]
        + [pl.BlockSpec((f_in, H), lambda i: (0, 0)),
           pl.BlockSpec((f_in, H), lambda i: (0, 0)),
           pl.BlockSpec((1, H), lambda i: (0, 0))]
    )
    if final:
        in_specs += (
            [pl.BlockSpec((R, FC), lambda i: (i, 0)) for _ in range(8)]
            + [pl.BlockSpec((3 * H, C), lambda i: (0, 0)),
               pl.BlockSpec((1, C), lambda i: (0, 0))]
        )
        out_specs = pl.BlockSpec((R, C), lambda i: (i, 0))
        out_shape = jax.ShapeDtypeStruct((N, C), jnp.float32)
    else:
        out_specs = [pl.BlockSpec((R, FC), lambda i: (i, 0)) for _ in range(H // FC)]
        out_shape = [jax.ShapeDtypeStruct((N, FC), jnp.float32) for _ in range(H // FC)]

    def body(*refs):
        parts = refs[:npass_in]
        cnt = refs[npass_in]
        o = npass_in + 1
        xins = refs[o:o + npass_in]
        o += npass_in
        Wl, Wr, b = refs[o:o + 3]
        o += 3
        if final:
            hprev = refs[o:o + 8]
            Wlin, blin = refs[o + 8:o + 10]
            o += 10
            out = refs[o]
        else:
            outs = refs[o:]

        s = [pr[0] + pr[1] for pr in parts]
        mean = jnp.concatenate(s, axis=1)
        cb = cnt[...]
        deg = cb[0, :, 0:1] + cb[1, :, 0:1]
        mean = mean / jnp.maximum(deg, 1.0)
        x = jnp.concatenate([xr[...] for xr in xins], axis=1)
        h = (jnp.dot(mean, Wl[...], preferred_element_type=jnp.float32)
             + jnp.dot(x, Wr[...], preferred_element_type=jnp.float32)
             + b[...])
        nrm = jnp.sqrt(jnp.sum(h * h, axis=1, keepdims=True))
        h = jnp.maximum(h / jnp.maximum(nrm, 1e-12), 0.0)
        if final:
            cat = jnp.concatenate([hr[...] for hr in hprev] + [h], axis=1)
            out[...] = (jnp.dot(cat, Wlin[...], preferred_element_type=jnp.float32)
                        + blin[...])
        else:
            for q in range(H // FC):
                outs[q][...] = h[:, q * FC:(q + 1) * FC]

    return pl.pallas_call(body, grid=grid, in_specs=in_specs,
                          out_specs=out_specs, out_shape=out_shape)


_AGG1 = _sc_agg(F_IN // FC, with_counts=True)
_AGG2 = _sc_agg(H // FC, with_counts=False)
_DENSE1 = _dense_layer(F_IN // FC, F_IN, final=False)
_DENSE2 = _dense_layer(H // FC, H, final=False)
_DENSE3 = _dense_layer(H // FC, H, final=True)


def kernel(x, edge_index, W1l, b1l, W1r, W2l, b2l, W2r, W3l, b3l, W3r, Wlin, blin):
    src = edge_index[0].astype(jnp.int32)
    dst = edge_index[1].astype(jnp.int32)
    pad = EP - E
    srcp = jnp.concatenate([src, jnp.zeros((pad,), jnp.int32)]).reshape(NCHUNKS, CH)
    dstp = jnp.concatenate([dst, jnp.full((pad,), DUMP, jnp.int32)]).reshape(NCHUNKS, CH)
    x_c = [x[:, i * FC:(i + 1) * FC] for i in range(F_IN // FC)]
    b1 = b1l.reshape(1, H)
    b2 = b2l.reshape(1, H)
    b3 = b3l.reshape(1, H)
    bl = blin.reshape(1, C)

    p10, p11, cnt = _AGG1(x_c[0], x_c[1], srcp, dstp)
    h1c = _DENSE1(p10, p11, cnt, x_c[0], x_c[1], W1l, W1r, b1)
    p2 = _AGG2(h1c[0], h1c[1], h1c[2], h1c[3], srcp, dstp)
    h2c = _DENSE2(p2[0], p2[1], p2[2], p2[3], cnt,
                  h1c[0], h1c[1], h1c[2], h1c[3], W2l, W2r, b2)
    p3 = _AGG2(h2c[0], h2c[1], h2c[2], h2c[3], srcp, dstp)
    final = _DENSE3(p3[0], p3[1], p3[2], p3[3], cnt,
                    h2c[0], h2c[1], h2c[2], h2c[3], W3l, W3r, b3,
                    h1c[0], h1c[1], h1c[2], h1c[3],
                    h2c[0], h2c[1], h2c[2], h2c[3], Wlin, bl)
    return final


# speedup vs baseline: 2.2834x; 1.1166x over previous
import functools

import jax
import jax.numpy as jnp
from jax import lax
from jax.experimental import pallas as pl
from jax.experimental.pallas import tpu as pltpu
from jax.experimental.pallas import tpu_sc as plsc

N = 10000
F_IN = 256
H = 512
C = 16
E = 160000

FC = 128
NROWS = 10112
ROWS_PT = NROWS // 16
DUMP = N
CH = 64
NCH0 = 120
NCH1 = 40
EP = 16 * (NCH0 + NCH1) * CH
NCHUNKS = EP // CH
CW = 16
R = 400
_ZSTEPS = tuple((k, 32) for k in range(0, 608, 32)) + ((608, 24),)


def _sc_agg(npass, with_counts):
    mesh = plsc.VectorSubcoreMesh(core_axis_name="c", subcore_axis_name="s",
                                  num_cores=2, num_subcores=16)
    outs = [jax.ShapeDtypeStruct((2, NROWS, FC), jnp.float32) for _ in range(npass)]
    if with_counts:
        outs.append(jax.ShapeDtypeStruct((2, NROWS, CW), jnp.float32))
    scratch = [
        pltpu.VMEM((max(NCH0, NCH1), CH), jnp.int32),
        pltpu.VMEM((max(NCH0, NCH1), CH), jnp.int32),
        pltpu.VMEM((CH, FC), jnp.float32),
        pltpu.VMEM((CH, FC), jnp.float32),
        pltpu.VMEM((32, FC), jnp.float32),
        pltpu.VMEM_SHARED((NROWS, FC), jnp.float32),
        pltpu.SemaphoreType.DMA,
        pltpu.SemaphoreType.DMA,
        pltpu.SemaphoreType.DMA,
        pltpu.SemaphoreType.DMA,
    ]
    if with_counts:
        scratch += [
            pltpu.VMEM((CH, CW), jnp.float32),
            pltpu.VMEM((32, CW), jnp.float32),
            pltpu.VMEM_SHARED((NROWS, CW), jnp.float32),
            pltpu.SemaphoreType.DMA,
        ]

    def body(*refs):
        xs = refs[:npass]
        src2d, dst2d = refs[npass], refs[npass + 1]
        o = npass + 2
        outs_r = refs[o:o + npass]
        o += npass
        if with_counts:
            cnt_out = refs[o]
            o += 1
        src_v, dst_v, rows0, rows1, zbuf, acc, g0, g1, s0, s1 = refs[o:o + 10]
        o += 10
        if with_counts:
            ones, zbuf16, cnt_acc, semc = refs[o:o + 4]

        ci = lax.axis_index("c")
        si = lax.axis_index("s")
        base = si * ROWS_PT

        zero16 = jnp.zeros((16,), jnp.float32)
        one16 = jnp.ones((16,), jnp.float32)

        @pl.loop(0, 32)
        def _fill(i):
            @pl.loop(0, FC // 16)
            def _fz(j):
                zbuf[i, pl.ds(j * 16, 16)] = zero16
            if with_counts:
                zbuf16[i, pl.ds(0, 16)] = zero16
        if with_counts:
            @pl.loop(0, CH)
            def _fo(i):
                ones[i, pl.ds(0, 16)] = one16

        @pl.when(ci == 0)
        def _stage0():
            pltpu.sync_copy(src2d.at[pl.ds(si * NCH0, NCH0)],
                            src_v.at[pl.ds(0, NCH0)])
            pltpu.sync_copy(dst2d.at[pl.ds(si * NCH0, NCH0)],
                            dst_v.at[pl.ds(0, NCH0)])

        @pl.when(ci == 1)
        def _stage1():
            pltpu.sync_copy(src2d.at[pl.ds(16 * NCH0 + si * NCH1, NCH1)],
                            src_v.at[pl.ds(0, NCH1)])
            pltpu.sync_copy(dst2d.at[pl.ds(16 * NCH0 + si * NCH1, NCH1)],
                            dst_v.at[pl.ds(0, NCH1)])

        if with_counts:
            for k, sz in _ZSTEPS:
                pltpu.sync_copy(zbuf16.at[pl.ds(0, sz)],
                                cnt_acc.at[pl.ds(base + k, sz)])

        def _run_chunks(p, nch):
            @pl.loop(0, nch // 2)
            def _chunks(i):
                j0 = 2 * i
                j1 = j0 + 1
                pltpu.async_copy(xs[p].at[src_v.at[j1]], rows1, g1)
                pltpu.make_async_copy(xs[p].at[src_v.at[j0]], rows0, g0).wait()
                pltpu.async_copy(rows0, acc.at[dst_v.at[j0]], s0, add=True)
                if with_counts and p == 0:
                    pltpu.async_copy(ones, cnt_acc.at[dst_v.at[j0]], semc, add=True)
                pltpu.make_async_copy(xs[p].at[src_v.at[j1]], rows1, g1).wait()
                pltpu.async_copy(rows1, acc.at[dst_v.at[j1]], s1, add=True)
                if with_counts and p == 0:
                    pltpu.async_copy(ones, cnt_acc.at[dst_v.at[j1]], semc, add=True)
                pltpu.make_async_copy(rows0, acc.at[dst_v.at[j0]], s0).wait()

                @pl.when(i + 1 < nch // 2)
                def _pref():
                    pltpu.async_copy(xs[p].at[src_v.at[j0 + 2]], rows0, g0)

                pltpu.make_async_copy(rows1, acc.at[dst_v.at[j1]], s1).wait()

            if with_counts and p == 0:
                @pl.loop(0, nch)
                def _drain(i):
                    pltpu.make_async_copy(ones, cnt_acc.at[dst_v.at[i]], semc).wait()

        for p in range(npass):
            pltpu.async_copy(xs[p].at[src_v.at[0]], rows0, g0)
            for k, sz in _ZSTEPS:
                pltpu.sync_copy(zbuf.at[pl.ds(0, sz)],
                                acc.at[pl.ds(base + k, sz)])
            plsc.subcore_barrier()

            @pl.when(ci == 0)
            def _c0():
                _run_chunks(p, NCH0)

            @pl.when(ci == 1)
            def _c1():
                _run_chunks(p, NCH1)

            plsc.subcore_barrier()
            pltpu.sync_copy(acc.at[pl.ds(base, ROWS_PT)],
                            outs_r[p].at[ci, pl.ds(base, ROWS_PT)])
            if with_counts and p == 0:
                pltpu.sync_copy(cnt_acc.at[pl.ds(base, ROWS_PT)],
                                cnt_out.at[ci, pl.ds(base, ROWS_PT)])

    return pl.kernel(body, out_type=outs, mesh=mesh, scratch_types=scratch,
                     compiler_params=pltpu.CompilerParams(use_tc_tiling_on_sc=False))


def _dense_layer(npass_in, f_in, final):
    grid = (N // R,)

    in_specs = (
        [pl.BlockSpec((2, R, FC), lambda i: (0, i, 0)) for _ in range(npass_in)]
        + [pl.BlockSpec((2, R, CW), lambda i: (0, i, 0))]
        + [pl.BlockSpec((R, FC), lambda i: (i, 0)) for _ in range(npass_in)]
        + [pl.BlockSpec((f_in, H), lambda i: (0, 0)),
           pl.BlockSpec((f_in, H), lambda i: (0, 0)),
           pl.BlockSpec((1, H), lambda i: (0, 0))]
    )
    if final:
        in_specs += (
            [pl.BlockSpec((R, FC), lambda i: (i, 0)) for _ in range(8)]
            + [pl.BlockSpec((3 * H, C), lambda i: (0, 0)),
               pl.BlockSpec((1, C), lambda i: (0, 0))]
        )
        out_specs = pl.BlockSpec((R, C), lambda i: (i, 0))
        out_shape = jax.ShapeDtypeStruct((N, C), jnp.float32)
    else:
        out_specs = [pl.BlockSpec((R, FC), lambda i: (i, 0)) for _ in range(H // FC)]
        out_shape = [jax.ShapeDtypeStruct((N, FC), jnp.float32) for _ in range(H // FC)]

    def body(*refs):
        parts = refs[:npass_in]
        cnt = refs[npass_in]
        o = npass_in + 1
        xins = refs[o:o + npass_in]
        o += npass_in
        Wl, Wr, b = refs[o:o + 3]
        o += 3
        if final:
            hprev = refs[o:o + 8]
            Wlin, blin = refs[o + 8:o + 10]
            o += 10
            out = refs[o]
        else:
            outs = refs[o:]

        s = [pr[0] + pr[1] for pr in parts]
        mean = jnp.concatenate(s, axis=1)
        cb = cnt[...]
        deg = cb[0, :, 0:1] + cb[1, :, 0:1]
        mean = mean / jnp.maximum(deg, 1.0)
        x = jnp.concatenate([xr[...] for xr in xins], axis=1)
        h = (jnp.dot(mean, Wl[...], preferred_element_type=jnp.float32)
             + jnp.dot(x, Wr[...], preferred_element_type=jnp.float32)
             + b[...])
        nrm = jnp.sqrt(jnp.sum(h * h, axis=1, keepdims=True))
        h = jnp.maximum(h / jnp.maximum(nrm, 1e-12), 0.0)
        if final:
            cat = jnp.concatenate([hr[...] for hr in hprev] + [h], axis=1)
            out[...] = (jnp.dot(cat, Wlin[...], preferred_element_type=jnp.float32)
                        + blin[...])
        else:
            for q in range(H // FC):
                outs[q][...] = h[:, q * FC:(q + 1) * FC]

    return pl.pallas_call(body, grid=grid, in_specs=in_specs,
                          out_specs=out_specs, out_shape=out_shape)


_AGG1 = _sc_agg(F_IN // FC, with_counts=True)
_AGG2 = _sc_agg(H // FC, with_counts=False)
_DENSE1 = _dense_layer(F_IN // FC, F_IN, final=False)
_DENSE2 = _dense_layer(H // FC, H, final=False)
_DENSE3 = _dense_layer(H // FC, H, final=True)


def kernel(x, edge_index, W1l, b1l, W1r, W2l, b2l, W2r, W3l, b3l, W3r, Wlin, blin):
    src = edge_index[0].astype(jnp.int32)
    dst = edge_index[1].astype(jnp.int32)
    pad = EP - E
    srcp = jnp.concatenate([src, jnp.zeros((pad,), jnp.int32)]).reshape(NCHUNKS, CH)
    dstp = jnp.concatenate([dst, jnp.full((pad,), DUMP, jnp.int32)]).reshape(NCHUNKS, CH)
    x_c = [x[:, i * FC:(i + 1) * FC] for i in range(F_IN // FC)]
    b1 = b1l.reshape(1, H)
    b2 = b2l.reshape(1, H)
    b3 = b3l.reshape(1, H)
    bl = blin.reshape(1, C)

    p10, p11, cnt = _AGG1(x_c[0], x_c[1], srcp, dstp)
    h1c = _DENSE1(p10, p11, cnt, x_c[0], x_c[1], W1l, W1r, b1)
    p2 = _AGG2(h1c[0], h1c[1], h1c[2], h1c[3], srcp, dstp)
    h2c = _DENSE2(p2[0], p2[1], p2[2], p2[3], cnt,
                  h1c[0], h1c[1], h1c[2], h1c[3], W2l, W2r, b2)
    p3 = _AGG2(h2c[0], h2c[1], h2c[2], h2c[3], srcp, dstp)
    final = _DENSE3(p3[0], p3[1], p3[2], p3[3], cnt,
                    h2c[0], h2c[1], h2c[2], h2c[3], W3l, W3r, b3,
                    h1c[0], h1c[1], h1c[2], h1c[3],
                    h2c[0], h2c[1], h2c[2], h2c[3], Wlin, bl)
    return final


# speedup vs baseline: 2.4297x; 1.0641x over previous
import functools

import jax
import jax.numpy as jnp
from jax import lax
from jax.experimental import pallas as pl
from jax.experimental.pallas import tpu as pltpu
from jax.experimental.pallas import tpu_sc as plsc

N = 10000
F_IN = 256
H = 512
C = 16
E = 160000

FC = 128
NROWS = 10112
ROWS_PT = NROWS // 16
DUMP = N
CH = 64
NCH0 = 134
NCH1 = 26
EP = 16 * (NCH0 + NCH1) * CH
NCHUNKS = EP // CH
CW = 16
R = 400
_ZSTEPS = tuple((k, 32) for k in range(0, 608, 32)) + ((608, 24),)


def _sc_agg(npass, with_counts):
    mesh = plsc.VectorSubcoreMesh(core_axis_name="c", subcore_axis_name="s",
                                  num_cores=2, num_subcores=16)
    outs = [jax.ShapeDtypeStruct((2, NROWS, FC), jnp.float32) for _ in range(npass)]
    if with_counts:
        outs.append(jax.ShapeDtypeStruct((2, NROWS, CW), jnp.float32))
    scratch = [
        pltpu.VMEM((max(NCH0, NCH1), CH), jnp.int32),
        pltpu.VMEM((max(NCH0, NCH1), CH), jnp.int32),
        pltpu.VMEM((CH, FC), jnp.float32),
        pltpu.VMEM((CH, FC), jnp.float32),
        pltpu.VMEM((32, FC), jnp.float32),
        pltpu.VMEM_SHARED((NROWS, FC), jnp.float32),
        pltpu.SemaphoreType.DMA,
        pltpu.SemaphoreType.DMA,
        pltpu.SemaphoreType.DMA,
        pltpu.SemaphoreType.DMA,
    ]
    if with_counts:
        scratch += [
            pltpu.VMEM((CH, CW), jnp.float32),
            pltpu.VMEM((32, CW), jnp.float32),
            pltpu.VMEM_SHARED((NROWS, CW), jnp.float32),
            pltpu.SemaphoreType.DMA,
        ]

    def body(*refs):
        xs = refs[:npass]
        src2d, dst2d = refs[npass], refs[npass + 1]
        o = npass + 2
        outs_r = refs[o:o + npass]
        o += npass
        if with_counts:
            cnt_out = refs[o]
            o += 1
        src_v, dst_v, rows0, rows1, zbuf, acc, g0, g1, s0, s1 = refs[o:o + 10]
        o += 10
        if with_counts:
            ones, zbuf16, cnt_acc, semc = refs[o:o + 4]

        ci = lax.axis_index("c")
        si = lax.axis_index("s")
        base = si * ROWS_PT

        zero16 = jnp.zeros((16,), jnp.float32)
        one16 = jnp.ones((16,), jnp.float32)

        @pl.loop(0, 32)
        def _fill(i):
            @pl.loop(0, FC // 16)
            def _fz(j):
                zbuf[i, pl.ds(j * 16, 16)] = zero16
            if with_counts:
                zbuf16[i, pl.ds(0, 16)] = zero16
        if with_counts:
            @pl.loop(0, CH)
            def _fo(i):
                ones[i, pl.ds(0, 16)] = one16

        @pl.when(ci == 0)
        def _stage0():
            pltpu.sync_copy(src2d.at[pl.ds(si * NCH0, NCH0)],
                            src_v.at[pl.ds(0, NCH0)])
            pltpu.sync_copy(dst2d.at[pl.ds(si * NCH0, NCH0)],
                            dst_v.at[pl.ds(0, NCH0)])

        @pl.when(ci == 1)
        def _stage1():
            pltpu.sync_copy(src2d.at[pl.ds(16 * NCH0 + si * NCH1, NCH1)],
                            src_v.at[pl.ds(0, NCH1)])
            pltpu.sync_copy(dst2d.at[pl.ds(16 * NCH0 + si * NCH1, NCH1)],
                            dst_v.at[pl.ds(0, NCH1)])

        if with_counts:
            for k, sz in _ZSTEPS:
                pltpu.sync_copy(zbuf16.at[pl.ds(0, sz)],
                                cnt_acc.at[pl.ds(base + k, sz)])

        def _run_chunks(p, nch):
            @pl.loop(0, nch // 2)
            def _chunks(i):
                j0 = 2 * i
                j1 = j0 + 1
                pltpu.async_copy(xs[p].at[src_v.at[j1]], rows1, g1)
                pltpu.make_async_copy(xs[p].at[src_v.at[j0]], rows0, g0).wait()
                pltpu.async_copy(rows0, acc.at[dst_v.at[j0]], s0, add=True)
                if with_counts and p == 0:
                    pltpu.async_copy(ones, cnt_acc.at[dst_v.at[j0]], semc, add=True)
                pltpu.make_async_copy(xs[p].at[src_v.at[j1]], rows1, g1).wait()
                pltpu.async_copy(rows1, acc.at[dst_v.at[j1]], s1, add=True)
                if with_counts and p == 0:
                    pltpu.async_copy(ones, cnt_acc.at[dst_v.at[j1]], semc, add=True)
                pltpu.make_async_copy(rows0, acc.at[dst_v.at[j0]], s0).wait()

                @pl.when(i + 1 < nch // 2)
                def _pref():
                    pltpu.async_copy(xs[p].at[src_v.at[j0 + 2]], rows0, g0)

                pltpu.make_async_copy(rows1, acc.at[dst_v.at[j1]], s1).wait()

            if with_counts and p == 0:
                @pl.loop(0, nch)
                def _drain(i):
                    pltpu.make_async_copy(ones, cnt_acc.at[dst_v.at[i]], semc).wait()

        for p in range(npass):
            pltpu.async_copy(xs[p].at[src_v.at[0]], rows0, g0)
            for k, sz in _ZSTEPS:
                pltpu.sync_copy(zbuf.at[pl.ds(0, sz)],
                                acc.at[pl.ds(base + k, sz)])
            plsc.subcore_barrier()

            @pl.when(ci == 0)
            def _c0():
                _run_chunks(p, NCH0)

            @pl.when(ci == 1)
            def _c1():
                _run_chunks(p, NCH1)

            plsc.subcore_barrier()
            pltpu.sync_copy(acc.at[pl.ds(base, ROWS_PT)],
                            outs_r[p].at[ci, pl.ds(base, ROWS_PT)])
            if with_counts and p == 0:
                pltpu.sync_copy(cnt_acc.at[pl.ds(base, ROWS_PT)],
                                cnt_out.at[ci, pl.ds(base, ROWS_PT)])

    return pl.kernel(body, out_type=outs, mesh=mesh, scratch_types=scratch,
                     compiler_params=pltpu.CompilerParams(use_tc_tiling_on_sc=False))


def _dense_layer(npass_in, f_in, final):
    grid = (N // R,)

    in_specs = (
        [pl.BlockSpec((2, R, FC), lambda i: (0, i, 0)) for _ in range(npass_in)]
        + [pl.BlockSpec((2, R, CW), lambda i: (0, i, 0))]
        + [pl.BlockSpec((R, FC), lambda i: (i, 0)) for _ in range(npass_in)]
        + [pl.BlockSpec((f_in, H), lambda i: (0, 0)),
           pl.BlockSpec((f_in, H), lambda i: (0, 0)),
           pl.BlockSpec((1, H), lambda i: (0, 0))]
    )
    if final:
        in_specs += (
            [pl.BlockSpec((R, FC), lambda i: (i, 0)) for _ in range(8)]
            + [pl.BlockSpec((3 * H, C), lambda i: (0, 0)),
               pl.BlockSpec((1, C), lambda i: (0, 0))]
        )
        out_specs = pl.BlockSpec((R, C), lambda i: (i, 0))
        out_shape = jax.ShapeDtypeStruct((N, C), jnp.float32)
    else:
        out_specs = [pl.BlockSpec((R, FC), lambda i: (i, 0)) for _ in range(H // FC)]
        out_shape = [jax.ShapeDtypeStruct((N, FC), jnp.float32) for _ in range(H // FC)]

    def body(*refs):
        parts = refs[:npass_in]
        cnt = refs[npass_in]
        o = npass_in + 1
        xins = refs[o:o + npass_in]
        o += npass_in
        Wl, Wr, b = refs[o:o + 3]
        o += 3
        if final:
            hprev = refs[o:o + 8]
            Wlin, blin = refs[o + 8:o + 10]
            o += 10
            out = refs[o]
        else:
            outs = refs[o:]

        s = [pr[0] + pr[1] for pr in parts]
        mean = jnp.concatenate(s, axis=1)
        cb = cnt[...]
        deg = cb[0, :, 0:1] + cb[1, :, 0:1]
        mean = mean / jnp.maximum(deg, 1.0)
        x = jnp.concatenate([xr[...] for xr in xins], axis=1)
        h = (jnp.dot(mean, Wl[...], preferred_element_type=jnp.float32)
             + jnp.dot(x, Wr[...], preferred_element_type=jnp.float32)
             + b[...])
        nrm = jnp.sqrt(jnp.sum(h * h, axis=1, keepdims=True))
        h = jnp.maximum(h / jnp.maximum(nrm, 1e-12), 0.0)
        if final:
            cat = jnp.concatenate([hr[...] for hr in hprev] + [h], axis=1)
            out[...] = (jnp.dot(cat, Wlin[...], preferred_element_type=jnp.float32)
                        + blin[...])
        else:
            for q in range(H // FC):
                outs[q][...] = h[:, q * FC:(q + 1) * FC]

    return pl.pallas_call(body, grid=grid, in_specs=in_specs,
                          out_specs=out_specs, out_shape=out_shape)


_AGG1 = _sc_agg(F_IN // FC, with_counts=True)
_AGG2 = _sc_agg(H // FC, with_counts=False)
_DENSE1 = _dense_layer(F_IN // FC, F_IN, final=False)
_DENSE2 = _dense_layer(H // FC, H, final=False)
_DENSE3 = _dense_layer(H // FC, H, final=True)


def kernel(x, edge_index, W1l, b1l, W1r, W2l, b2l, W2r, W3l, b3l, W3r, Wlin, blin):
    src = edge_index[0].astype(jnp.int32)
    dst = edge_index[1].astype(jnp.int32)
    pad = EP - E
    srcp = jnp.concatenate([src, jnp.zeros((pad,), jnp.int32)]).reshape(NCHUNKS, CH)
    dstp = jnp.concatenate([dst, jnp.full((pad,), DUMP, jnp.int32)]).reshape(NCHUNKS, CH)
    x_c = [x[:, i * FC:(i + 1) * FC] for i in range(F_IN // FC)]
    b1 = b1l.reshape(1, H)
    b2 = b2l.reshape(1, H)
    b3 = b3l.reshape(1, H)
    bl = blin.reshape(1, C)

    p10, p11, cnt = _AGG1(x_c[0], x_c[1], srcp, dstp)
    h1c = _DENSE1(p10, p11, cnt, x_c[0], x_c[1], W1l, W1r, b1)
    p2 = _AGG2(h1c[0], h1c[1], h1c[2], h1c[3], srcp, dstp)
    h2c = _DENSE2(p2[0], p2[1], p2[2], p2[3], cnt,
                  h1c[0], h1c[1], h1c[2], h1c[3], W2l, W2r, b2)
    p3 = _AGG2(h2c[0], h2c[1], h2c[2], h2c[3], srcp, dstp)
    final = _DENSE3(p3[0], p3[1], p3[2], p3[3], cnt,
                    h2c[0], h2c[1], h2c[2], h2c[3], W3l, W3r, b3,
                    h1c[0], h1c[1], h1c[2], h1c[3],
                    h2c[0], h2c[1], h2c[2], h2c[3], Wlin, bl)
    return final


# speedup vs baseline: 5.2782x; 2.1724x over previous
import functools

import jax
import jax.numpy as jnp
from jax import lax
from jax.experimental import pallas as pl
from jax.experimental.pallas import tpu as pltpu
from jax.experimental.pallas import tpu_sc as plsc

N = 10000
F_IN = 256
H = 512
C = 16
E = 160000

FC = 128
NROWS = 10112
ROWS_PT = NROWS // 16
DUMP = N
CH = 64
NCH0 = 80
NCH1 = 80
EP = 16 * (NCH0 + NCH1) * CH
NCHUNKS = EP // CH
CW = 16
R = 400
_ZSTEPS = tuple((k, 32) for k in range(0, 608, 32)) + ((608, 24),)


def _sc_agg(npass, with_counts):
    mesh = plsc.VectorSubcoreMesh(core_axis_name="c", subcore_axis_name="s",
                                  num_cores=2, num_subcores=16)
    outs = [jax.ShapeDtypeStruct((2, NROWS, FC), jnp.float32) for _ in range(npass)]
    if with_counts:
        outs.append(jax.ShapeDtypeStruct((2, NROWS, CW), jnp.float32))
    scratch = [
        pltpu.VMEM((max(NCH0, NCH1), CH), jnp.int32),
        pltpu.VMEM((max(NCH0, NCH1), CH), jnp.int32),
        pltpu.VMEM((CH, FC), jnp.float32),
        pltpu.VMEM((CH, FC), jnp.float32),
        pltpu.VMEM((32, FC), jnp.float32),
        pltpu.VMEM_SHARED((NROWS, FC), jnp.float32),
        pltpu.SemaphoreType.DMA,
        pltpu.SemaphoreType.DMA,
        pltpu.SemaphoreType.DMA,
        pltpu.SemaphoreType.DMA,
    ]
    if with_counts:
        scratch += [
            pltpu.VMEM((CH, CW), jnp.float32),
            pltpu.VMEM((32, CW), jnp.float32),
            pltpu.VMEM_SHARED((NROWS, CW), jnp.float32),
            pltpu.SemaphoreType.DMA,
        ]

    def body(*refs):
        xs = refs[:npass]
        src2d, dst2d = refs[npass], refs[npass + 1]
        o = npass + 2
        outs_r = refs[o:o + npass]
        o += npass
        if with_counts:
            cnt_out = refs[o]
            o += 1
        src_v, dst_v, rows0, rows1, zbuf, acc, g0, g1, s0, s1 = refs[o:o + 10]
        o += 10
        if with_counts:
            ones, zbuf16, cnt_acc, semc = refs[o:o + 4]

        ci = lax.axis_index("c")
        si = lax.axis_index("s")
        base = si * ROWS_PT

        zero16 = jnp.zeros((16,), jnp.float32)
        one16 = jnp.ones((16,), jnp.float32)

        @pl.loop(0, 32)
        def _fill(i):
            @pl.loop(0, FC // 16)
            def _fz(j):
                zbuf[i, pl.ds(j * 16, 16)] = zero16
            if with_counts:
                zbuf16[i, pl.ds(0, 16)] = zero16
        if with_counts:
            @pl.loop(0, CH)
            def _fo(i):
                ones[i, pl.ds(0, 16)] = one16

        @pl.when(ci == 0)
        def _stage0():
            pltpu.sync_copy(src2d.at[pl.ds(si * NCH0, NCH0)],
                            src_v.at[pl.ds(0, NCH0)])
            pltpu.sync_copy(dst2d.at[pl.ds(si * NCH0, NCH0)],
                            dst_v.at[pl.ds(0, NCH0)])

        @pl.when(ci == 1)
        def _stage1():
            pltpu.sync_copy(src2d.at[pl.ds(16 * NCH0 + si * NCH1, NCH1)],
                            src_v.at[pl.ds(0, NCH1)])
            pltpu.sync_copy(dst2d.at[pl.ds(16 * NCH0 + si * NCH1, NCH1)],
                            dst_v.at[pl.ds(0, NCH1)])

        if with_counts:
            for k, sz in _ZSTEPS:
                pltpu.sync_copy(zbuf16.at[pl.ds(0, sz)],
                                cnt_acc.at[pl.ds(base + k, sz)])

        def _run_chunks(p, nch):
            @pl.loop(0, nch // 2)
            def _chunks(i):
                j0 = 2 * i
                j1 = j0 + 1
                pltpu.async_copy(xs[p].at[src_v.at[j1]], rows1, g1)
                pltpu.make_async_copy(xs[p].at[src_v.at[j0]], rows0, g0).wait()
                pltpu.async_copy(rows0, acc.at[dst_v.at[j0]], s0, add=True)
                if with_counts and p == 0:
                    pltpu.async_copy(ones, cnt_acc.at[dst_v.at[j0]], semc, add=True)
                pltpu.make_async_copy(xs[p].at[src_v.at[j1]], rows1, g1).wait()
                pltpu.async_copy(rows1, acc.at[dst_v.at[j1]], s1, add=True)
                if with_counts and p == 0:
                    pltpu.async_copy(ones, cnt_acc.at[dst_v.at[j1]], semc, add=True)
                pltpu.make_async_copy(rows0, acc.at[dst_v.at[j0]], s0).wait()

                @pl.when(i + 1 < nch // 2)
                def _pref():
                    pltpu.async_copy(xs[p].at[src_v.at[j0 + 2]], rows0, g0)

                pltpu.make_async_copy(rows1, acc.at[dst_v.at[j1]], s1).wait()

            if with_counts and p == 0:
                @pl.loop(0, nch)
                def _drain(i):
                    pltpu.make_async_copy(ones, cnt_acc.at[dst_v.at[i]], semc).wait()

        for p in range(npass):
            pltpu.async_copy(xs[p].at[src_v.at[0]], rows0, g0)
            for k, sz in _ZSTEPS:
                pltpu.sync_copy(zbuf.at[pl.ds(0, sz)],
                                acc.at[pl.ds(base + k, sz)])
            plsc.subcore_barrier()

            @pl.when(ci == 0)
            def _c0():
                _run_chunks(p, NCH0)

            @pl.when(ci == 1)
            def _c1():
                _run_chunks(p, NCH1)

            plsc.subcore_barrier()
            pltpu.sync_copy(acc.at[pl.ds(base, ROWS_PT)],
                            outs_r[p].at[ci, pl.ds(base, ROWS_PT)])
            if with_counts and p == 0:
                pltpu.sync_copy(cnt_acc.at[pl.ds(base, ROWS_PT)],
                                cnt_out.at[ci, pl.ds(base, ROWS_PT)])

    return pl.kernel(body, out_type=outs, mesh=mesh, scratch_types=scratch,
                     compiler_params=pltpu.CompilerParams(use_tc_tiling_on_sc=False))


def _dense_layer(npass_in, f_in, final):
    grid = (N // R,)

    in_specs = (
        [pl.BlockSpec((2, R, FC), lambda i: (0, i, 0)) for _ in range(npass_in)]
        + [pl.BlockSpec((2, R, CW), lambda i: (0, i, 0))]
        + [pl.BlockSpec((R, FC), lambda i: (i, 0)) for _ in range(npass_in)]
        + [pl.BlockSpec((f_in, H), lambda i: (0, 0)),
           pl.BlockSpec((f_in, H), lambda i: (0, 0)),
           pl.BlockSpec((1, H), lambda i: (0, 0))]
    )
    if final:
        in_specs += (
            [pl.BlockSpec((R, FC), lambda i: (i, 0)) for _ in range(8)]
            + [pl.BlockSpec((3 * H, C), lambda i: (0, 0)),
               pl.BlockSpec((1, C), lambda i: (0, 0))]
        )
        out_specs = pl.BlockSpec((R, C), lambda i: (i, 0))
        out_shape = jax.ShapeDtypeStruct((N, C), jnp.float32)
    else:
        out_specs = [pl.BlockSpec((R, FC), lambda i: (i, 0)) for _ in range(H // FC)]
        out_shape = [jax.ShapeDtypeStruct((N, FC), jnp.float32) for _ in range(H // FC)]

    def body(*refs):
        parts = refs[:npass_in]
        cnt = refs[npass_in]
        o = npass_in + 1
        xins = refs[o:o + npass_in]
        o += npass_in
        Wl, Wr, b = refs[o:o + 3]
        o += 3
        if final:
            hprev = refs[o:o + 8]
            Wlin, blin = refs[o + 8:o + 10]
            o += 10
            out = refs[o]
        else:
            outs = refs[o:]

        s = [pr[0] + pr[1] for pr in parts]
        mean = jnp.concatenate(s, axis=1)
        cb = cnt[...]
        deg = cb[0, :, 0:1] + cb[1, :, 0:1]
        mean = mean / jnp.maximum(deg, 1.0)
        x = jnp.concatenate([xr[...] for xr in xins], axis=1)
        h = (jnp.dot(mean, Wl[...], preferred_element_type=jnp.float32)
             + jnp.dot(x, Wr[...], preferred_element_type=jnp.float32)
             + b[...])
        nrm = jnp.sqrt(jnp.sum(h * h, axis=1, keepdims=True))
        h = jnp.maximum(h / jnp.maximum(nrm, 1e-12), 0.0)
        if final:
            cat = jnp.concatenate([hr[...] for hr in hprev] + [h], axis=1)
            out[...] = (jnp.dot(cat, Wlin[...], preferred_element_type=jnp.float32)
                        + blin[...])
        else:
            for q in range(H // FC):
                outs[q][...] = h[:, q * FC:(q + 1) * FC]

    return pl.pallas_call(body, grid=grid, in_specs=in_specs,
                          out_specs=out_specs, out_shape=out_shape)


_AGG1 = _sc_agg(F_IN // FC, with_counts=True)
_AGG2 = _sc_agg(H // FC, with_counts=False)
_DENSE1 = _dense_layer(F_IN // FC, F_IN, final=False)
_DENSE2 = _dense_layer(H // FC, H, final=False)
_DENSE3 = _dense_layer(H // FC, H, final=True)


def kernel(x, edge_index, W1l, b1l, W1r, W2l, b2l, W2r, W3l, b3l, W3r, Wlin, blin):
    src = edge_index[0].astype(jnp.int32)
    dst = edge_index[1].astype(jnp.int32)
    pad = EP - E
    fill = jnp.arange(pad, dtype=jnp.int32)
    srcp = jnp.concatenate([src, fill % N]).reshape(NCHUNKS, CH)
    dstp = jnp.concatenate([dst, DUMP + fill % (NROWS - N)]).reshape(NCHUNKS, CH)
    x_c = [x[:, i * FC:(i + 1) * FC] for i in range(F_IN // FC)]
    b1 = b1l.reshape(1, H)
    b2 = b2l.reshape(1, H)
    b3 = b3l.reshape(1, H)
    bl = blin.reshape(1, C)

    p10, p11, cnt = _AGG1(x_c[0], x_c[1], srcp, dstp)
    h1c = _DENSE1(p10, p11, cnt, x_c[0], x_c[1], W1l, W1r, b1)
    p2 = _AGG2(h1c[0], h1c[1], h1c[2], h1c[3], srcp, dstp)
    h2c = _DENSE2(p2[0], p2[1], p2[2], p2[3], cnt,
                  h1c[0], h1c[1], h1c[2], h1c[3], W2l, W2r, b2)
    p3 = _AGG2(h2c[0], h2c[1], h2c[2], h2c[3], srcp, dstp)
    final = _DENSE3(p3[0], p3[1], p3[2], p3[3], cnt,
                    h2c[0], h2c[1], h2c[2], h2c[3], W3l, W3r, b3,
                    h1c[0], h1c[1], h1c[2], h1c[3],
                    h2c[0], h2c[1], h2c[2], h2c[3], Wlin, bl)
    return final


# speedup vs baseline: 5.8128x; 1.1013x over previous
import functools

import jax
import jax.numpy as jnp
from jax import lax
from jax.experimental import pallas as pl
from jax.experimental.pallas import tpu as pltpu
from jax.experimental.pallas import tpu_sc as plsc

N = 10000
F_IN = 256
H = 512
C = 16
E = 160000

FC = 128
NROWS = 10112
ROWS_PT = NROWS // 16
DUMP = N
CH = 64
NCH0 = 80
NCH1 = 80
EP = 16 * (NCH0 + NCH1) * CH
NCHUNKS = EP // CH
CW = 16
R = 400
_ZSTEPS = tuple((k, 32) for k in range(0, 608, 32)) + ((608, 24),)


def _sc_agg(npass, with_counts, ch, nch0, nch1):
    mesh = plsc.VectorSubcoreMesh(core_axis_name="c", subcore_axis_name="s",
                                  num_cores=2, num_subcores=16)
    outs = [jax.ShapeDtypeStruct((2, NROWS, FC), jnp.float32) for _ in range(npass)]
    if with_counts:
        outs.append(jax.ShapeDtypeStruct((2, NROWS, CW), jnp.float32))
    scratch = [
        pltpu.VMEM((max(nch0, nch1), ch), jnp.int32),
        pltpu.VMEM((max(nch0, nch1), ch), jnp.int32),
        pltpu.VMEM((ch, FC), jnp.float32),
        pltpu.VMEM((ch, FC), jnp.float32),
        pltpu.VMEM((32, FC), jnp.float32),
        pltpu.VMEM_SHARED((NROWS, FC), jnp.float32),
        pltpu.SemaphoreType.DMA,
        pltpu.SemaphoreType.DMA,
        pltpu.SemaphoreType.DMA,
        pltpu.SemaphoreType.DMA,
    ]
    if with_counts:
        scratch += [
            pltpu.VMEM((ch, CW), jnp.float32),
            pltpu.VMEM((32, CW), jnp.float32),
            pltpu.VMEM_SHARED((NROWS, CW), jnp.float32),
            pltpu.SemaphoreType.DMA,
        ]

    def body(*refs):
        xs = refs[:npass]
        src2d, dst2d = refs[npass], refs[npass + 1]
        o = npass + 2
        outs_r = refs[o:o + npass]
        o += npass
        if with_counts:
            cnt_out = refs[o]
            o += 1
        src_v, dst_v, rows0, rows1, zbuf, acc, g0, g1, s0, s1 = refs[o:o + 10]
        o += 10
        if with_counts:
            ones, zbuf16, cnt_acc, semc = refs[o:o + 4]

        ci = lax.axis_index("c")
        si = lax.axis_index("s")
        base = si * ROWS_PT

        zero16 = jnp.zeros((16,), jnp.float32)
        one16 = jnp.ones((16,), jnp.float32)

        @pl.loop(0, 32)
        def _fill(i):
            @pl.loop(0, FC // 16)
            def _fz(j):
                zbuf[i, pl.ds(j * 16, 16)] = zero16
            if with_counts:
                zbuf16[i, pl.ds(0, 16)] = zero16
        if with_counts:
            @pl.loop(0, ch)
            def _fo(i):
                ones[i, pl.ds(0, 16)] = one16

        @pl.when(ci == 0)
        def _stage0():
            pltpu.sync_copy(src2d.at[pl.ds(si * nch0, nch0)],
                            src_v.at[pl.ds(0, nch0)])
            pltpu.sync_copy(dst2d.at[pl.ds(si * nch0, nch0)],
                            dst_v.at[pl.ds(0, nch0)])

        @pl.when(ci == 1)
        def _stage1():
            pltpu.sync_copy(src2d.at[pl.ds(16 * nch0 + si * nch1, nch1)],
                            src_v.at[pl.ds(0, nch1)])
            pltpu.sync_copy(dst2d.at[pl.ds(16 * nch0 + si * nch1, nch1)],
                            dst_v.at[pl.ds(0, nch1)])

        if with_counts:
            for k, sz in _ZSTEPS:
                pltpu.sync_copy(zbuf16.at[pl.ds(0, sz)],
                                cnt_acc.at[pl.ds(base + k, sz)])

        def _run_chunks(p, nch):
            @pl.loop(0, nch // 2)
            def _chunks(i):
                j0 = 2 * i
                j1 = j0 + 1
                pltpu.async_copy(xs[p].at[src_v.at[j1]], rows1, g1)
                pltpu.make_async_copy(xs[p].at[src_v.at[j0]], rows0, g0).wait()
                pltpu.async_copy(rows0, acc.at[dst_v.at[j0]], s0, add=True)
                if with_counts and p == 0:
                    pltpu.async_copy(ones, cnt_acc.at[dst_v.at[j0]], semc, add=True)
                pltpu.make_async_copy(xs[p].at[src_v.at[j1]], rows1, g1).wait()
                pltpu.async_copy(rows1, acc.at[dst_v.at[j1]], s1, add=True)
                if with_counts and p == 0:
                    pltpu.async_copy(ones, cnt_acc.at[dst_v.at[j1]], semc, add=True)
                pltpu.make_async_copy(rows0, acc.at[dst_v.at[j0]], s0).wait()

                @pl.when(i + 1 < nch // 2)
                def _pref():
                    pltpu.async_copy(xs[p].at[src_v.at[j0 + 2]], rows0, g0)

                pltpu.make_async_copy(rows1, acc.at[dst_v.at[j1]], s1).wait()

            if with_counts and p == 0:
                @pl.loop(0, nch)
                def _drain(i):
                    pltpu.make_async_copy(ones, cnt_acc.at[dst_v.at[i]], semc).wait()

        for p in range(npass):
            pltpu.async_copy(xs[p].at[src_v.at[0]], rows0, g0)
            for k, sz in _ZSTEPS:
                pltpu.sync_copy(zbuf.at[pl.ds(0, sz)],
                                acc.at[pl.ds(base + k, sz)])
            plsc.subcore_barrier()

            @pl.when(ci == 0)
            def _c0():
                _run_chunks(p, nch0)

            @pl.when(ci == 1)
            def _c1():
                _run_chunks(p, nch1)

            plsc.subcore_barrier()
            pltpu.sync_copy(acc.at[pl.ds(base, ROWS_PT)],
                            outs_r[p].at[ci, pl.ds(base, ROWS_PT)])
            if with_counts and p == 0:
                pltpu.sync_copy(cnt_acc.at[pl.ds(base, ROWS_PT)],
                                cnt_out.at[ci, pl.ds(base, ROWS_PT)])

    return pl.kernel(body, out_type=outs, mesh=mesh, scratch_types=scratch,
                     compiler_params=pltpu.CompilerParams(use_tc_tiling_on_sc=False))


def _dense_layer(npass_in, f_in, final):
    grid = (N // R,)

    in_specs = (
        [pl.BlockSpec((2, R, FC), lambda i: (0, i, 0)) for _ in range(npass_in)]
        + [pl.BlockSpec((2, R, CW), lambda i: (0, i, 0))]
        + [pl.BlockSpec((R, FC), lambda i: (i, 0)) for _ in range(npass_in)]
        + [pl.BlockSpec((f_in, H), lambda i: (0, 0)),
           pl.BlockSpec((f_in, H), lambda i: (0, 0)),
           pl.BlockSpec((1, H), lambda i: (0, 0))]
    )
    if final:
        in_specs += (
            [pl.BlockSpec((R, FC), lambda i: (i, 0)) for _ in range(8)]
            + [pl.BlockSpec((3 * H, C), lambda i: (0, 0)),
               pl.BlockSpec((1, C), lambda i: (0, 0))]
        )
        out_specs = pl.BlockSpec((R, C), lambda i: (i, 0))
        out_shape = jax.ShapeDtypeStruct((N, C), jnp.float32)
    else:
        out_specs = [pl.BlockSpec((R, FC), lambda i: (i, 0)) for _ in range(H // FC)]
        out_shape = [jax.ShapeDtypeStruct((N, FC), jnp.float32) for _ in range(H // FC)]

    def body(*refs):
        parts = refs[:npass_in]
        cnt = refs[npass_in]
        o = npass_in + 1
        xins = refs[o:o + npass_in]
        o += npass_in
        Wl, Wr, b = refs[o:o + 3]
        o += 3
        if final:
            hprev = refs[o:o + 8]
            Wlin, blin = refs[o + 8:o + 10]
            o += 10
            out = refs[o]
        else:
            outs = refs[o:]

        s = [pr[0] + pr[1] for pr in parts]
        mean = jnp.concatenate(s, axis=1)
        cb = cnt[...]
        deg = cb[0, :, 0:1] + cb[1, :, 0:1]
        mean = mean / jnp.maximum(deg, 1.0)
        x = jnp.concatenate([xr[...] for xr in xins], axis=1)
        h = (jnp.dot(mean, Wl[...], preferred_element_type=jnp.float32)
             + jnp.dot(x, Wr[...], preferred_element_type=jnp.float32)
             + b[...])
        nrm = jnp.sqrt(jnp.sum(h * h, axis=1, keepdims=True))
        h = jnp.maximum(h / jnp.maximum(nrm, 1e-12), 0.0)
        if final:
            cat = jnp.concatenate([hr[...] for hr in hprev] + [h], axis=1)
            out[...] = (jnp.dot(cat, Wlin[...], preferred_element_type=jnp.float32)
                        + blin[...])
        else:
            for q in range(H // FC):
                outs[q][...] = h[:, q * FC:(q + 1) * FC]

    return pl.pallas_call(body, grid=grid, in_specs=in_specs,
                          out_specs=out_specs, out_shape=out_shape)


_AGG1 = _sc_agg(F_IN // FC, with_counts=True, ch=64, nch0=80, nch1=80)
_AGG2 = _sc_agg(H // FC, with_counts=False, ch=128, nch0=40, nch1=40)
_DENSE1 = _dense_layer(F_IN // FC, F_IN, final=False)
_DENSE2 = _dense_layer(H // FC, H, final=False)
_DENSE3 = _dense_layer(H // FC, H, final=True)


def kernel(x, edge_index, W1l, b1l, W1r, W2l, b2l, W2r, W3l, b3l, W3r, Wlin, blin):
    src = edge_index[0].astype(jnp.int32)
    dst = edge_index[1].astype(jnp.int32)
    pad = EP - E
    fill = jnp.arange(pad, dtype=jnp.int32)
    src_flat = jnp.concatenate([src, fill % N])
    dst_flat = jnp.concatenate([dst, DUMP + fill % (NROWS - N)])
    srcp = src_flat.reshape(EP // 64, 64)
    dstp = dst_flat.reshape(EP // 64, 64)
    srcp2 = src_flat.reshape(EP // 128, 128)
    dstp2 = dst_flat.reshape(EP // 128, 128)
    x_c = [x[:, i * FC:(i + 1) * FC] for i in range(F_IN // FC)]
    b1 = b1l.reshape(1, H)
    b2 = b2l.reshape(1, H)
    b3 = b3l.reshape(1, H)
    bl = blin.reshape(1, C)

    p10, p11, cnt = _AGG1(x_c[0], x_c[1], srcp, dstp)
    h1c = _DENSE1(p10, p11, cnt, x_c[0], x_c[1], W1l, W1r, b1)
    p2 = _AGG2(h1c[0], h1c[1], h1c[2], h1c[3], srcp2, dstp2)
    h2c = _DENSE2(p2[0], p2[1], p2[2], p2[3], cnt,
                  h1c[0], h1c[1], h1c[2], h1c[3], W2l, W2r, b2)
    p3 = _AGG2(h2c[0], h2c[1], h2c[2], h2c[3], srcp2, dstp2)
    final = _DENSE3(p3[0], p3[1], p3[2], p3[3], cnt,
                    h2c[0], h2c[1], h2c[2], h2c[3], W3l, W3r, b3,
                    h1c[0], h1c[1], h1c[2], h1c[3],
                    h2c[0], h2c[1], h2c[2], h2c[3], Wlin, bl)
    return final


# speedup vs baseline: 5.8660x; 1.0092x over previous
import functools

import jax
import jax.numpy as jnp
from jax import lax
from jax.experimental import pallas as pl
from jax.experimental.pallas import tpu as pltpu
from jax.experimental.pallas import tpu_sc as plsc

N = 10000
F_IN = 256
H = 512
C = 16
E = 160000

FC = 128
NROWS = 10112
ROWS_PT = NROWS // 16
DUMP = N
CH = 64
NCH0 = 80
NCH1 = 80
EP = 16 * (NCH0 + NCH1) * CH
NCHUNKS = EP // CH
CW = 16
R = 400
_ZSTEPS = tuple((k, 32) for k in range(0, 608, 32)) + ((608, 24),)


def _sc_agg(npass, with_counts, ch, nch0, nch1):
    mesh = plsc.VectorSubcoreMesh(core_axis_name="c", subcore_axis_name="s",
                                  num_cores=2, num_subcores=16)
    outs = [jax.ShapeDtypeStruct((2, NROWS, FC), jnp.float32) for _ in range(npass)]
    if with_counts:
        outs.append(jax.ShapeDtypeStruct((2, NROWS, CW), jnp.float32))
    scratch = [
        pltpu.VMEM((max(nch0, nch1), ch), jnp.int32),
        pltpu.VMEM((max(nch0, nch1), ch), jnp.int32),
        pltpu.VMEM((ch, FC), jnp.float32),
        pltpu.VMEM((ch, FC), jnp.float32),
        pltpu.VMEM((32, FC), jnp.float32),
        pltpu.VMEM_SHARED((NROWS, FC), jnp.float32),
        pltpu.SemaphoreType.DMA,
        pltpu.SemaphoreType.DMA,
        pltpu.SemaphoreType.DMA,
        pltpu.SemaphoreType.DMA,
    ]
    if with_counts:
        scratch += [
            pltpu.VMEM((ch, CW), jnp.float32),
            pltpu.VMEM((32, CW), jnp.float32),
            pltpu.VMEM_SHARED((NROWS, CW), jnp.float32),
            pltpu.SemaphoreType.DMA,
        ]

    def body(*refs):
        xs = refs[:npass]
        src2d, dst2d = refs[npass], refs[npass + 1]
        o = npass + 2
        outs_r = refs[o:o + npass]
        o += npass
        if with_counts:
            cnt_out = refs[o]
            o += 1
        src_v, dst_v, rows0, rows1, zbuf, acc, g0, g1, s0, s1 = refs[o:o + 10]
        o += 10
        if with_counts:
            ones, zbuf16, cnt_acc, semc = refs[o:o + 4]

        ci = lax.axis_index("c")
        si = lax.axis_index("s")
        base = si * ROWS_PT

        zero16 = jnp.zeros((16,), jnp.float32)
        one16 = jnp.ones((16,), jnp.float32)

        @pl.loop(0, 32)
        def _fill(i):
            @pl.loop(0, FC // 16)
            def _fz(j):
                zbuf[i, pl.ds(j * 16, 16)] = zero16
            if with_counts:
                zbuf16[i, pl.ds(0, 16)] = zero16
        if with_counts:
            @pl.loop(0, ch)
            def _fo(i):
                ones[i, pl.ds(0, 16)] = one16

        @pl.when(ci == 0)
        def _stage0():
            pltpu.sync_copy(src2d.at[pl.ds(si * nch0, nch0)],
                            src_v.at[pl.ds(0, nch0)])
            pltpu.sync_copy(dst2d.at[pl.ds(si * nch0, nch0)],
                            dst_v.at[pl.ds(0, nch0)])

        @pl.when(ci == 1)
        def _stage1():
            pltpu.sync_copy(src2d.at[pl.ds(16 * nch0 + si * nch1, nch1)],
                            src_v.at[pl.ds(0, nch1)])
            pltpu.sync_copy(dst2d.at[pl.ds(16 * nch0 + si * nch1, nch1)],
                            dst_v.at[pl.ds(0, nch1)])

        if with_counts:
            for k, sz in _ZSTEPS:
                pltpu.sync_copy(zbuf16.at[pl.ds(0, sz)],
                                cnt_acc.at[pl.ds(base + k, sz)])

        def _run_chunks(p, nch):
            @pl.loop(0, nch // 2)
            def _chunks(i):
                j0 = 2 * i
                j1 = j0 + 1
                pltpu.async_copy(xs[p].at[src_v.at[j1]], rows1, g1)
                pltpu.make_async_copy(xs[p].at[src_v.at[j0]], rows0, g0).wait()
                pltpu.async_copy(rows0, acc.at[dst_v.at[j0]], s0, add=True)
                if with_counts and p == 0:
                    pltpu.async_copy(ones, cnt_acc.at[dst_v.at[j0]], semc, add=True)
                pltpu.make_async_copy(xs[p].at[src_v.at[j1]], rows1, g1).wait()
                pltpu.async_copy(rows1, acc.at[dst_v.at[j1]], s1, add=True)
                if with_counts and p == 0:
                    pltpu.async_copy(ones, cnt_acc.at[dst_v.at[j1]], semc, add=True)
                pltpu.make_async_copy(rows0, acc.at[dst_v.at[j0]], s0).wait()

                @pl.when(i + 1 < nch // 2)
                def _pref():
                    pltpu.async_copy(xs[p].at[src_v.at[j0 + 2]], rows0, g0)

                pltpu.make_async_copy(rows1, acc.at[dst_v.at[j1]], s1).wait()

            if with_counts and p == 0:
                @pl.loop(0, nch)
                def _drain(i):
                    pltpu.make_async_copy(ones, cnt_acc.at[dst_v.at[i]], semc).wait()

        for p in range(npass):
            pltpu.async_copy(xs[p].at[src_v.at[0]], rows0, g0)
            for k, sz in _ZSTEPS:
                pltpu.async_copy(zbuf.at[pl.ds(0, sz)],
                                 acc.at[pl.ds(base + k, sz)], s0)
            for k, sz in _ZSTEPS:
                pltpu.make_async_copy(zbuf.at[pl.ds(0, sz)],
                                      acc.at[pl.ds(base + k, sz)], s0).wait()
            plsc.subcore_barrier()

            @pl.when(ci == 0)
            def _c0():
                _run_chunks(p, nch0)

            @pl.when(ci == 1)
            def _c1():
                _run_chunks(p, nch1)

            plsc.subcore_barrier()
            pltpu.sync_copy(acc.at[pl.ds(base, ROWS_PT)],
                            outs_r[p].at[ci, pl.ds(base, ROWS_PT)])
            if with_counts and p == 0:
                pltpu.sync_copy(cnt_acc.at[pl.ds(base, ROWS_PT)],
                                cnt_out.at[ci, pl.ds(base, ROWS_PT)])

    return pl.kernel(body, out_type=outs, mesh=mesh, scratch_types=scratch,
                     compiler_params=pltpu.CompilerParams(use_tc_tiling_on_sc=False))


def _dense_layer(npass_in, f_in, final):
    grid = (N // R,)

    in_specs = (
        [pl.BlockSpec((2, R, FC), lambda i: (0, i, 0)) for _ in range(npass_in)]
        + [pl.BlockSpec((2, R, CW), lambda i: (0, i, 0))]
        + [pl.BlockSpec((R, FC), lambda i: (i, 0)) for _ in range(npass_in)]
        + [pl.BlockSpec((f_in, H), lambda i: (0, 0)),
           pl.BlockSpec((f_in, H), lambda i: (0, 0)),
           pl.BlockSpec((1, H), lambda i: (0, 0))]
    )
    if final:
        in_specs += (
            [pl.BlockSpec((R, FC), lambda i: (i, 0)) for _ in range(8)]
            + [pl.BlockSpec((3 * H, C), lambda i: (0, 0)),
               pl.BlockSpec((1, C), lambda i: (0, 0))]
        )
        out_specs = pl.BlockSpec((R, C), lambda i: (i, 0))
        out_shape = jax.ShapeDtypeStruct((N, C), jnp.float32)
    else:
        out_specs = [pl.BlockSpec((R, FC), lambda i: (i, 0)) for _ in range(H // FC)]
        out_shape = [jax.ShapeDtypeStruct((N, FC), jnp.float32) for _ in range(H // FC)]

    def body(*refs):
        parts = refs[:npass_in]
        cnt = refs[npass_in]
        o = npass_in + 1
        xins = refs[o:o + npass_in]
        o += npass_in
        Wl, Wr, b = refs[o:o + 3]
        o += 3
        if final:
            hprev = refs[o:o + 8]
            Wlin, blin = refs[o + 8:o + 10]
            o += 10
            out = refs[o]
        else:
            outs = refs[o:]

        s = [pr[0] + pr[1] for pr in parts]
        mean = jnp.concatenate(s, axis=1)
        cb = cnt[...]
        deg = cb[0, :, 0:1] + cb[1, :, 0:1]
        mean = mean / jnp.maximum(deg, 1.0)
        x = jnp.concatenate([xr[...] for xr in xins], axis=1)
        h = (jnp.dot(mean, Wl[...], preferred_element_type=jnp.float32)
             + jnp.dot(x, Wr[...], preferred_element_type=jnp.float32)
             + b[...])
        nrm = jnp.sqrt(jnp.sum(h * h, axis=1, keepdims=True))
        h = jnp.maximum(h / jnp.maximum(nrm, 1e-12), 0.0)
        if final:
            cat = jnp.concatenate([hr[...] for hr in hprev] + [h], axis=1)
            out[...] = (jnp.dot(cat, Wlin[...], preferred_element_type=jnp.float32)
                        + blin[...])
        else:
            for q in range(H // FC):
                outs[q][...] = h[:, q * FC:(q + 1) * FC]

    return pl.pallas_call(body, grid=grid, in_specs=in_specs,
                          out_specs=out_specs, out_shape=out_shape)


_AGG1 = _sc_agg(F_IN // FC, with_counts=True, ch=64, nch0=80, nch1=80)
_AGG2 = _sc_agg(H // FC, with_counts=False, ch=128, nch0=40, nch1=40)
_DENSE1 = _dense_layer(F_IN // FC, F_IN, final=False)
_DENSE2 = _dense_layer(H // FC, H, final=False)
_DENSE3 = _dense_layer(H // FC, H, final=True)


def kernel(x, edge_index, W1l, b1l, W1r, W2l, b2l, W2r, W3l, b3l, W3r, Wlin, blin):
    src = edge_index[0].astype(jnp.int32)
    dst = edge_index[1].astype(jnp.int32)
    pad = EP - E
    fill = jnp.arange(pad, dtype=jnp.int32)
    src_flat = jnp.concatenate([src, fill % N])
    dst_flat = jnp.concatenate([dst, DUMP + fill % (NROWS - N)])
    srcp = src_flat.reshape(EP // 64, 64)
    dstp = dst_flat.reshape(EP // 64, 64)
    srcp2 = src_flat.reshape(EP // 128, 128)
    dstp2 = dst_flat.reshape(EP // 128, 128)
    x_c = [x[:, i * FC:(i + 1) * FC] for i in range(F_IN // FC)]
    b1 = b1l.reshape(1, H)
    b2 = b2l.reshape(1, H)
    b3 = b3l.reshape(1, H)
    bl = blin.reshape(1, C)

    p10, p11, cnt = _AGG1(x_c[0], x_c[1], srcp, dstp)
    h1c = _DENSE1(p10, p11, cnt, x_c[0], x_c[1], W1l, W1r, b1)
    p2 = _AGG2(h1c[0], h1c[1], h1c[2], h1c[3], srcp2, dstp2)
    h2c = _DENSE2(p2[0], p2[1], p2[2], p2[3], cnt,
                  h1c[0], h1c[1], h1c[2], h1c[3], W2l, W2r, b2)
    p3 = _AGG2(h2c[0], h2c[1], h2c[2], h2c[3], srcp2, dstp2)
    final = _DENSE3(p3[0], p3[1], p3[2], p3[3], cnt,
                    h2c[0], h2c[1], h2c[2], h2c[3], W3l, W3r, b3,
                    h1c[0], h1c[1], h1c[2], h1c[3],
                    h2c[0], h2c[1], h2c[2], h2c[3], Wlin, bl)
    return final
